# baseline (speedup 1.0000x reference)
import functools

import jax
import jax.numpy as jnp
from jax import lax
from jax.experimental import pallas as pl
from jax.experimental.pallas import tpu as pltpu
from jax.experimental.pallas import tpu_sc as plsc

_HIDDEN = 128
_NW = 32
_CHUNK = 64
_NBUF = 8
_LEAD = 4


@functools.cache
def _make_emb_kernel(n_tokens, vocab):
    chunks_per_w = n_tokens // (_NW * _CHUNK)
    assert chunks_per_w % _NBUF == 0 and chunks_per_w >= 2 * _NBUF
    mesh = plsc.VectorSubcoreMesh(core_axis_name="c", subcore_axis_name="s")

    @functools.partial(
        pl.kernel,
        mesh=mesh,
        out_type=jax.ShapeDtypeStruct((n_tokens, _HIDDEN), jnp.float32),
        scratch_types=[
            pltpu.VMEM((chunks_per_w, _CHUNK), jnp.int32),
            pltpu.VMEM((_NBUF, _CHUNK, _HIDDEN), jnp.float32),
        ]
        + [pltpu.SemaphoreType.DMA] * (2 * _NBUF),
    )
    def emb(idx_hbm, table_hbm, out_hbm, idx_v, rows_v, *sems):
        gs, ss = sems[:_NBUF], sems[_NBUF:]
        wid = lax.axis_index("s") * 2 + lax.axis_index("c")
        base = wid * (chunks_per_w * _CHUNK)
        pltpu.sync_copy(idx_hbm.at[wid], idx_v)

        def g_copy(j, b):
            return pltpu.make_async_copy(
                table_hbm.at[idx_v.at[j]], rows_v.at[b], gs[b])

        def s_copy(j, b):
            return pltpu.make_async_copy(
                rows_v.at[b], out_hbm.at[pl.ds(base + j * _CHUNK, _CHUNK)], ss[b])

        for j in range(_LEAD):
            g_copy(j, j % _NBUF).start()

        for b in range(_NBUF):
            j = b
            g_copy(j, b).wait()
            s_copy(j, b).start()
            if j >= _LEAD:
                s_copy(j - _LEAD, (b - _LEAD) % _NBUF).wait()
            g_copy(j + _LEAD, (b + _LEAD) % _NBUF).start()

        def group(g, carry):
            j0 = g * _NBUF
            for b in range(_NBUF):
                j = j0 + b
                g_copy(j, b).wait()
                s_copy(j, b).start()
                s_copy(j - _LEAD, (b - _LEAD) % _NBUF).wait()
                g_copy(j + _LEAD, (b + _LEAD) % _NBUF).start()
            return carry

        lax.fori_loop(1, chunks_per_w // _NBUF - 1, group, 0)

        j0 = chunks_per_w - _NBUF
        for b in range(_NBUF):
            j = j0 + b
            g_copy(j, b).wait()
            s_copy(j, b).start()
            s_copy(j - _LEAD, (b - _LEAD) % _NBUF).wait()
            if j + _LEAD < chunks_per_w:
                g_copy(j + _LEAD, (b + _LEAD) % _NBUF).start()

        for b in range(_NBUF - _LEAD, _NBUF):
            s_copy(j0 + b, b).wait()

    return emb


def kernel(input_ids, embed_tokens_weight):
    batch, seq = input_ids.shape
    vocab, hidden = embed_tokens_weight.shape
    n_tokens = batch * seq
    idx = input_ids.astype(jnp.int32).reshape(_NW, n_tokens // (_NW * _CHUNK), _CHUNK)
    out = _make_emb_kernel(n_tokens, vocab)(idx, embed_tokens_weight)
    return out.reshape(batch, seq, hidden)


# speedup vs baseline: 1.0055x; 1.0055x over previous
import functools

import jax
import jax.numpy as jnp
from jax import lax
from jax.experimental import pallas as pl
from jax.experimental.pallas import tpu as pltpu
from jax.experimental.pallas import tpu_sc as plsc

_HIDDEN = 128
_NW = 32
_CHUNK = 128
_NBUF = 4
_LEAD = 2


@functools.cache
def _make_emb_kernel(n_tokens, vocab):
    chunks_per_w = n_tokens // (_NW * _CHUNK)
    assert chunks_per_w % _NBUF == 0 and chunks_per_w >= 2 * _NBUF
    mesh = plsc.VectorSubcoreMesh(core_axis_name="c", subcore_axis_name="s")

    @functools.partial(
        pl.kernel,
        mesh=mesh,
        out_type=jax.ShapeDtypeStruct((n_tokens, _HIDDEN), jnp.float32),
        scratch_types=[
            pltpu.VMEM((chunks_per_w, _CHUNK), jnp.int32),
            pltpu.VMEM((_NBUF, _CHUNK, _HIDDEN), jnp.float32),
        ]
        + [pltpu.SemaphoreType.DMA] * (2 * _NBUF),
    )
    def emb(idx_hbm, table_hbm, out_hbm, idx_v, rows_v, *sems):
        gs, ss = sems[:_NBUF], sems[_NBUF:]
        wid = lax.axis_index("s") * 2 + lax.axis_index("c")
        base = wid * (chunks_per_w * _CHUNK)
        pltpu.sync_copy(idx_hbm.at[wid], idx_v)

        def g_copy(j, b):
            return pltpu.make_async_copy(
                table_hbm.at[idx_v.at[j]], rows_v.at[b], gs[b])

        def s_copy(j, b):
            return pltpu.make_async_copy(
                rows_v.at[b], out_hbm.at[pl.ds(base + j * _CHUNK, _CHUNK)], ss[b])

        for j in range(_LEAD):
            g_copy(j, j % _NBUF).start()

        for b in range(_NBUF):
            j = b
            if j >= _LEAD:
                s_copy(j - _LEAD, (b - _LEAD) % _NBUF).wait()
            g_copy(j + _LEAD, (b + _LEAD) % _NBUF).start()
            g_copy(j, b).wait()
            s_copy(j, b).start()

        def group(g, carry):
            j0 = g * _NBUF
            for b in range(_NBUF):
                j = j0 + b
                s_copy(j - _LEAD, (b - _LEAD) % _NBUF).wait()
                g_copy(j + _LEAD, (b + _LEAD) % _NBUF).start()
                g_copy(j, b).wait()
                s_copy(j, b).start()
            return carry

        lax.fori_loop(1, chunks_per_w // _NBUF - 1, group, 0)

        j0 = chunks_per_w - _NBUF
        for b in range(_NBUF):
            j = j0 + b
            s_copy(j - _LEAD, (b - _LEAD) % _NBUF).wait()
            if j + _LEAD < chunks_per_w:
                g_copy(j + _LEAD, (b + _LEAD) % _NBUF).start()
            g_copy(j, b).wait()
            s_copy(j, b).start()

        for b in range(_NBUF - _LEAD, _NBUF):
            s_copy(j0 + b, b).wait()

    return emb


def kernel(input_ids, embed_tokens_weight):
    batch, seq = input_ids.shape
    vocab, hidden = embed_tokens_weight.shape
    n_tokens = batch * seq
    idx = input_ids.astype(jnp.int32).reshape(_NW, n_tokens // (_NW * _CHUNK), _CHUNK)
    out = _make_emb_kernel(n_tokens, vocab)(idx, embed_tokens_weight)
    return out.reshape(batch, seq, hidden)
